# Initial kernel scaffold; baseline (speedup 1.0000x reference)
import jax
import jax.numpy as jnp
from jax.experimental import pallas as pl


def kernel(x):
    raise NotImplementedError("write your pallas kernel here")


# speedup vs baseline: 28.0228x; 28.0228x over previous
import jax
import jax.numpy as jnp
from jax import lax
from jax.experimental import pallas as pl
from jax.experimental.pallas import tpu as pltpu

_ROWS, _COLS = 128, 32768
_N = _ROWS * _COLS
_K = int(_N * 0.9)
_MIN32 = -2147483648
_MAX32 = 2147483647


def _threshold_body(x_ref, out_ref):
    u = lax.bitcast_convert_type(x_ref[...], jnp.int32)
    skey = jnp.where(u >= 0, u, ~(u & _MAX32))

    def body(i, p_u):
        bit = (1 << (31 - i)).astype(jnp.int32)
        cand_u = p_u | bit
        cand_s = cand_u ^ _MIN32
        cnt = jnp.sum((skey >= cand_s).astype(jnp.int32))
        return jnp.where(cnt >= _K, cand_u, p_u)

    p_u = lax.fori_loop(0, 32, body, jnp.int32(0), unroll=False)
    out_ref[0, 0] = p_u


def _mask_body(t_ref, x_ref, out_ref):
    ts = t_ref[0, 0] ^ _MIN32
    ubits = jnp.where(ts >= 0, ts, _MIN32 | (~ts))
    t = lax.bitcast_convert_type(ubits, jnp.float32)
    xv = x_ref[...]
    m = (xv >= t).astype(jnp.float32)
    out_ref[...] = xv + (m - xv)


def kernel(x):
    t_key = pl.pallas_call(
        _threshold_body,
        out_shape=jax.ShapeDtypeStruct((1, 1), jnp.int32),
        out_specs=pl.BlockSpec(memory_space=pltpu.SMEM),
        compiler_params=pltpu.CompilerParams(
            vmem_limit_bytes=100 * 1024 * 1024,
        ),
    )(x)

    block_rows = 8
    grid = _ROWS // block_rows
    out = pl.pallas_call(
        _mask_body,
        grid=(grid,),
        in_specs=[
            pl.BlockSpec(memory_space=pltpu.SMEM),
            pl.BlockSpec((block_rows, _COLS), lambda i: (i, 0)),
        ],
        out_specs=pl.BlockSpec((block_rows, _COLS), lambda i: (i, 0)),
        out_shape=jax.ShapeDtypeStruct((_ROWS, _COLS), jnp.float32),
    )(t_key, x)
    return out
